# Initial kernel scaffold; baseline (speedup 1.0000x reference)
import jax
import jax.numpy as jnp
from jax.experimental import pallas as pl


def kernel(features, points):
    raise NotImplementedError("write your pallas kernel here")


# speedup vs baseline: 2.0589x; 2.0589x over previous
import functools

import jax
import jax.numpy as jnp
from jax import lax
from jax.experimental import pallas as pl
from jax.experimental.pallas import tpu as pltpu
from jax.experimental.pallas import tpu_sc as plsc

_N = 2048
_B = 4
_K = 16
_L = 16
_NW = 32
_ROWS_PER_W = (_B * _N) // _NW
_WORKERS_PER_BATCH = _N // _ROWS_PER_W
_CHUNKS = _N // _L


def _knn_kernel(pts_hbm, out_hbm, table_v, out_v):
  wid = lax.axis_index("c") * 16 + lax.axis_index("s")
  batch = wid // _WORKERS_PER_BATCH
  part = wid % _WORKERS_PER_BATCH
  row0 = part * _ROWS_PER_W

  pltpu.sync_copy(pts_hbm.at[batch], table_v)

  lane = lax.iota(jnp.int32, _L)
  inf = jnp.full((_L,), jnp.inf, dtype=jnp.float32)
  zero_i = jnp.zeros((_L,), dtype=jnp.int32)

  def chunk_body(c, state):
    a_k, a_v, b_k, b_v, thr = state
    base = c * _L
    px = table_v[0, pl.ds(base, _L)]
    py = table_v[1, pl.ds(base, _L)]
    pz = table_v[2, pl.ds(base, _L)]
    qx, qy, qz = state_q[0], state_q[1], state_q[2]
    dx = px - qx
    dy = py - qy
    dz = pz - qz
    d2 = dx * dx + dy * dy + dz * dz

    def do_merge(ops):
      a_k, a_v, b_k, b_v, _ = ops
      idxv = lane + base
      c_k, c_v = plsc.sort_key_val(d2, idxv)
      r_k = lax.rev(c_k, (0,))
      r_v = lax.rev(c_v, (0,))
      m = r_k < b_k
      lo_k = jnp.where(m, r_k, b_k)
      lo_v = jnp.where(m, r_v, b_v)
      lo_k, lo_v = plsc.sort_key_val(lo_k, lo_v)
      r_k = lax.rev(lo_k, (0,))
      r_v = lax.rev(lo_v, (0,))
      m = r_k < a_k
      na_k = jnp.where(m, r_k, a_k)
      na_v = jnp.where(m, r_v, a_v)
      nb_k = jnp.where(m, a_k, r_k)
      nb_v = jnp.where(m, a_v, r_v)
      na_k, na_v = plsc.sort_key_val(na_k, na_v)
      nb_k, nb_v = plsc.sort_key_val(nb_k, nb_v)
      nthr = nb_k.at[zero_i].get(mode="promise_in_bounds")
      return na_k, na_v, nb_k, nb_v, nthr

    qual = d2 < thr
    nqual = plsc.all_reduce_population_count(qual)
    return lax.cond(nqual[0] != 0, do_merge, lambda ops: ops,
                    (a_k, a_v, b_k, b_v, thr))

  def row_body(l, g):
    ql = jnp.broadcast_to(l, (_L,)).astype(jnp.int32)
    qx = state_q16[0].at[ql].get(mode="promise_in_bounds")
    qy = state_q16[1].at[ql].get(mode="promise_in_bounds")
    qz = state_q16[2].at[ql].get(mode="promise_in_bounds")
    state_q[0], state_q[1], state_q[2] = qx, qy, qz

    a_k, a_v, b_k, b_v, _ = lax.fori_loop(
        0, _CHUNKS, chunk_body, (inf, zero_i, inf, zero_i, inf))

    b0_k = b_k.at[zero_i].get(mode="promise_in_bounds")
    b0_v = b_v.at[zero_i].get(mode="promise_in_bounds")
    sel0 = lane == 0
    f_k = jnp.where(sel0, b0_k, a_k)
    f_v = jnp.where(sel0, b0_v, a_v)
    _, f_v = plsc.sort_key_val(f_k, f_v)
    out_v[g * _L + l, :] = f_v
    return g

  def group_body(g, carry):
    state_q16[0] = table_v[0, pl.ds(row0 + g * _L, _L)]
    state_q16[1] = table_v[1, pl.ds(row0 + g * _L, _L)]
    state_q16[2] = table_v[2, pl.ds(row0 + g * _L, _L)]
    lax.fori_loop(0, _L, row_body, g)
    return carry

  state_q16 = [None, None, None]
  state_q = [None, None, None]
  lax.fori_loop(0, _ROWS_PER_W // _L, group_body, 0)

  pltpu.sync_copy(out_v, out_hbm.at[pl.ds(wid * _ROWS_PER_W, _ROWS_PER_W)])


def kernel(features, points):
  del features
  pts_t = jnp.transpose(points, (0, 2, 1))
  mesh = plsc.VectorSubcoreMesh(core_axis_name="c", subcore_axis_name="s")
  kfn = pl.kernel(
      _knn_kernel,
      out_type=jax.ShapeDtypeStruct((_B * _N, _K), jnp.int32),
      mesh=mesh,
      scratch_types=[
          pltpu.VMEM((3, _N), jnp.float32),
          pltpu.VMEM((_ROWS_PER_W, _K), jnp.int32),
      ],
      compiler_params=pltpu.CompilerParams(needs_layout_passes=False),
  )
  topk = kfn(pts_t).reshape(_B, _N, _K)
  batch_idx = jnp.broadcast_to(
      jnp.arange(_B, dtype=jnp.int32).reshape(_B, 1, 1, 1), (_B, _N, _K, 1))
  return jnp.concatenate([batch_idx, topk[..., None]], axis=3)


# speedup vs baseline: 4.3310x; 2.1036x over previous
import functools

import jax
import jax.numpy as jnp
from jax import lax
from jax.experimental import pallas as pl
from jax.experimental.pallas import tpu as pltpu
from jax.experimental.pallas import tpu_sc as plsc

_N = 2048
_B = 4
_K = 16
_L = 16
_NW = 32
_ROWS_PER_W = (_B * _N) // _NW
_WORKERS_PER_BATCH = _N // _ROWS_PER_W
_CHUNKS = _N // _L


def _knn_kernel(pts_hbm, out_hbm, table_v, out_v):
  wid = lax.axis_index("c") * 16 + lax.axis_index("s")
  batch = wid // _WORKERS_PER_BATCH
  part = wid % _WORKERS_PER_BATCH
  row0 = part * _ROWS_PER_W

  pltpu.sync_copy(pts_hbm.at[batch], table_v)

  lane = lax.iota(jnp.int32, _L)
  inf = jnp.full((_L,), jnp.inf, dtype=jnp.float32)
  zero_i = jnp.zeros((_L,), dtype=jnp.int32)

  def chunk_body(c, state):
    a_k, a_v, b_k, b_v, thr = state
    base = c * _L
    px = table_v[0, pl.ds(base, _L)]
    py = table_v[1, pl.ds(base, _L)]
    pz = table_v[2, pl.ds(base, _L)]
    qx, qy, qz = state_q[0], state_q[1], state_q[2]
    dx = px - qx
    dy = py - qy
    dz = pz - qz
    d2 = dx * dx + dy * dy + dz * dz

    def do_merge(ops):
      a_k, a_v, b_k, b_v, _ = ops
      idxv = lane + base
      c_k, c_v = plsc.sort_key_val(d2, idxv)
      r_k = lax.rev(c_k, (0,))
      r_v = lax.rev(c_v, (0,))
      m = r_k < b_k
      lo_k = jnp.where(m, r_k, b_k)
      lo_v = jnp.where(m, r_v, b_v)
      lo_k, lo_v = plsc.sort_key_val(lo_k, lo_v)
      r_k = lax.rev(lo_k, (0,))
      r_v = lax.rev(lo_v, (0,))
      m = r_k < a_k
      na_k = jnp.where(m, r_k, a_k)
      na_v = jnp.where(m, r_v, a_v)
      nb_k = jnp.where(m, a_k, r_k)
      nb_v = jnp.where(m, a_v, r_v)
      na_k, na_v = plsc.sort_key_val(na_k, na_v)
      nb_k, nb_v = plsc.sort_key_val(nb_k, nb_v)
      nthr = nb_k.at[zero_i].get(mode="promise_in_bounds")
      return na_k, na_v, nb_k, nb_v, nthr

    return do_merge((a_k, a_v, b_k, b_v, thr))

  def row_body(l, g):
    ql = jnp.broadcast_to(l, (_L,)).astype(jnp.int32)
    qx = state_q16[0].at[ql].get(mode="promise_in_bounds")
    qy = state_q16[1].at[ql].get(mode="promise_in_bounds")
    qz = state_q16[2].at[ql].get(mode="promise_in_bounds")
    state_q[0], state_q[1], state_q[2] = qx, qy, qz

    a_k, a_v, b_k, b_v, _ = lax.fori_loop(
        0, _CHUNKS, chunk_body, (inf, zero_i, inf, zero_i, inf))

    b0_k = b_k.at[zero_i].get(mode="promise_in_bounds")
    b0_v = b_v.at[zero_i].get(mode="promise_in_bounds")
    sel0 = lane == 0
    f_k = jnp.where(sel0, b0_k, a_k)
    f_v = jnp.where(sel0, b0_v, a_v)
    _, f_v = plsc.sort_key_val(f_k, f_v)
    out_v[g * _L + l, :] = f_v
    return g

  def group_body(g, carry):
    state_q16[0] = table_v[0, pl.ds(row0 + g * _L, _L)]
    state_q16[1] = table_v[1, pl.ds(row0 + g * _L, _L)]
    state_q16[2] = table_v[2, pl.ds(row0 + g * _L, _L)]
    lax.fori_loop(0, _L, row_body, g)
    return carry

  state_q16 = [None, None, None]
  state_q = [None, None, None]
  lax.fori_loop(0, _ROWS_PER_W // _L, group_body, 0)

  pltpu.sync_copy(out_v, out_hbm.at[pl.ds(wid * _ROWS_PER_W, _ROWS_PER_W)])


def kernel(features, points):
  del features
  pts_t = jnp.transpose(points, (0, 2, 1))
  mesh = plsc.VectorSubcoreMesh(core_axis_name="c", subcore_axis_name="s")
  kfn = pl.kernel(
      _knn_kernel,
      out_type=jax.ShapeDtypeStruct((_B * _N, _K), jnp.int32),
      mesh=mesh,
      scratch_types=[
          pltpu.VMEM((3, _N), jnp.float32),
          pltpu.VMEM((_ROWS_PER_W, _K), jnp.int32),
      ],
      compiler_params=pltpu.CompilerParams(needs_layout_passes=False),
  )
  topk = kfn(pts_t).reshape(_B, _N, _K)
  batch_idx = jnp.broadcast_to(
      jnp.arange(_B, dtype=jnp.int32).reshape(_B, 1, 1, 1), (_B, _N, _K, 1))
  return jnp.concatenate([batch_idx, topk[..., None]], axis=3)


# speedup vs baseline: 8.6757x; 2.0031x over previous
import functools

import jax
import jax.numpy as jnp
from jax import lax
from jax.experimental import pallas as pl
from jax.experimental.pallas import tpu as pltpu
from jax.experimental.pallas import tpu_sc as plsc

_N = 2048
_B = 4
_K = 16
_L = 16
_NW = 32
_ROWS_PER_W = (_B * _N) // _NW
_WORKERS_PER_BATCH = _N // _ROWS_PER_W
_CHUNKS = _N // _L


def _knn_kernel(pts_hbm, out_hbm, table_v, out_v):
  wid = lax.axis_index("c") * 16 + lax.axis_index("s")
  batch = wid // _WORKERS_PER_BATCH
  part = wid % _WORKERS_PER_BATCH
  row0 = part * _ROWS_PER_W

  pltpu.sync_copy(pts_hbm.at[batch], table_v)

  lane = lax.iota(jnp.int32, _L)
  inf = jnp.full((_L,), jnp.inf, dtype=jnp.float32)
  zero_i = jnp.zeros((_L,), dtype=jnp.int32)

  def chunk_body(c, state):
    a_k, a_v, e_k, e_v = state
    base = c * _L
    px = table_v[0, pl.ds(base, _L)]
    py = table_v[1, pl.ds(base, _L)]
    pz = table_v[2, pl.ds(base, _L)]
    qx, qy, qz = state_q[0], state_q[1], state_q[2]
    dx = px - qx
    dy = py - qy
    dz = pz - qz
    d2 = dx * dx + dy * dy + dz * dz
    idxv = lane + base
    c_k, c_v = plsc.sort_key_val(d2, idxv)
    r_k = lax.rev(c_k, (0,))
    r_v = lax.rev(c_v, (0,))
    m = r_k < a_k
    na_k = jnp.where(m, r_k, a_k)
    na_v = jnp.where(m, r_v, a_v)
    ev_k = jnp.where(m, a_k, r_k)
    ev_v = jnp.where(m, a_v, r_v)
    a_k, a_v = plsc.sort_key_val(na_k, na_v)
    m2 = ev_k < e_k
    e_k = jnp.where(m2, ev_k, e_k)
    e_v = jnp.where(m2, ev_v, e_v)
    return a_k, a_v, e_k, e_v

  def row_body(l, g):
    ql = jnp.broadcast_to(l, (_L,)).astype(jnp.int32)
    qx = state_q16[0].at[ql].get(mode="promise_in_bounds")
    qy = state_q16[1].at[ql].get(mode="promise_in_bounds")
    qz = state_q16[2].at[ql].get(mode="promise_in_bounds")
    state_q[0], state_q[1], state_q[2] = qx, qy, qz

    a_k, a_v, e_k, e_v = lax.fori_loop(
        0, _CHUNKS, chunk_body, (inf, zero_i, inf, zero_i))

    s_k, s_v = plsc.sort_key_val(e_k, e_v)
    b0_k = s_k.at[zero_i].get(mode="promise_in_bounds")
    b0_v = s_v.at[zero_i].get(mode="promise_in_bounds")
    sel0 = lane == 0
    f_k = jnp.where(sel0, b0_k, a_k)
    f_v = jnp.where(sel0, b0_v, a_v)
    _, f_v = plsc.sort_key_val(f_k, f_v)
    out_v[g * _L + l, :] = f_v
    return g

  def group_body(g, carry):
    state_q16[0] = table_v[0, pl.ds(row0 + g * _L, _L)]
    state_q16[1] = table_v[1, pl.ds(row0 + g * _L, _L)]
    state_q16[2] = table_v[2, pl.ds(row0 + g * _L, _L)]
    lax.fori_loop(0, _L, row_body, g)
    return carry

  state_q16 = [None, None, None]
  state_q = [None, None, None]
  lax.fori_loop(0, _ROWS_PER_W // _L, group_body, 0)

  pltpu.sync_copy(out_v, out_hbm.at[pl.ds(wid * _ROWS_PER_W, _ROWS_PER_W)])


def kernel(features, points):
  del features
  pts_t = jnp.transpose(points, (0, 2, 1))
  mesh = plsc.VectorSubcoreMesh(core_axis_name="c", subcore_axis_name="s")
  kfn = pl.kernel(
      _knn_kernel,
      out_type=jax.ShapeDtypeStruct((_B * _N, _K), jnp.int32),
      mesh=mesh,
      scratch_types=[
          pltpu.VMEM((3, _N), jnp.float32),
          pltpu.VMEM((_ROWS_PER_W, _K), jnp.int32),
      ],
      compiler_params=pltpu.CompilerParams(needs_layout_passes=False),
  )
  topk = kfn(pts_t).reshape(_B, _N, _K)
  batch_idx = jnp.broadcast_to(
      jnp.arange(_B, dtype=jnp.int32).reshape(_B, 1, 1, 1), (_B, _N, _K, 1))
  return jnp.concatenate([batch_idx, topk[..., None]], axis=3)


# speedup vs baseline: 15.2170x; 1.7540x over previous
import functools

import jax
import jax.numpy as jnp
from jax import lax
from jax.experimental import pallas as pl
from jax.experimental.pallas import tpu as pltpu
from jax.experimental.pallas import tpu_sc as plsc

_N = 2048
_B = 4
_K = 16
_L = 16
_NW = 32
_ROWS_PER_W = (_B * _N) // _NW
_WORKERS_PER_BATCH = _N // _ROWS_PER_W
_CHUNKS = _N // _L


def _knn_kernel(pts_hbm, out_hbm, table_v, out_v):
  wid = lax.axis_index("c") * 16 + lax.axis_index("s")
  batch = wid // _WORKERS_PER_BATCH
  part = wid % _WORKERS_PER_BATCH
  row0 = part * _ROWS_PER_W

  pltpu.sync_copy(pts_hbm.at[batch], table_v)

  lane = lax.iota(jnp.int32, _L)
  inf = jnp.full((_L,), jnp.inf, dtype=jnp.float32)
  zero_i = jnp.zeros((_L,), dtype=jnp.int32)

  _RPI = 2

  def chunk_body(c, state):
    base = c * _L
    px = table_v[0, pl.ds(base, _L)]
    py = table_v[1, pl.ds(base, _L)]
    pz = table_v[2, pl.ds(base, _L)]
    idxv = lane + base

    def upd(q, a_k, a_v, e_k, e_v):
      dx = px - q[0]
      dy = py - q[1]
      dz = pz - q[2]
      d2 = dx * dx + dy * dy + dz * dz
      c_k, c_v = plsc.sort_key_val(d2, idxv)
      r_k = lax.rev(c_k, (0,))
      r_v = lax.rev(c_v, (0,))
      m = r_k < a_k
      na_k = jnp.where(m, r_k, a_k)
      na_v = jnp.where(m, r_v, a_v)
      ev_k = jnp.where(m, a_k, r_k)
      ev_v = jnp.where(m, a_v, r_v)
      a_k, a_v = plsc.sort_key_val(na_k, na_v)
      m2 = ev_k < e_k
      return (a_k, a_v, jnp.where(m2, ev_k, e_k), jnp.where(m2, ev_v, e_v))

    out = []
    for i in range(_RPI):
      out.extend(upd(state_q[i], *state[4 * i:4 * i + 4]))
    return tuple(out)

  def finalize(a_k, a_v, e_k, e_v, row):
    s_k, s_v = plsc.sort_key_val(e_k, e_v)
    b0_k = s_k.at[zero_i].get(mode="promise_in_bounds")
    b0_v = s_v.at[zero_i].get(mode="promise_in_bounds")
    sel0 = lane == 0
    f_k = jnp.where(sel0, b0_k, a_k)
    f_v = jnp.where(sel0, b0_v, a_v)
    _, f_v = plsc.sort_key_val(f_k, f_v)
    out_v[row, :] = f_v

  def row_body(p, g):
    for i in range(_RPI):
      ql = jnp.broadcast_to(p * _RPI + i, (_L,)).astype(jnp.int32)
      state_q[i] = tuple(
          state_q16[c].at[ql].get(mode="promise_in_bounds") for c in range(3))

    init = (inf, zero_i, inf, zero_i) * _RPI
    res = lax.fori_loop(0, _CHUNKS, chunk_body, init)
    for i in range(_RPI):
      finalize(*res[4 * i:4 * i + 4], g * _L + p * _RPI + i)
    return g

  def group_body(g, carry):
    state_q16[0] = table_v[0, pl.ds(row0 + g * _L, _L)]
    state_q16[1] = table_v[1, pl.ds(row0 + g * _L, _L)]
    state_q16[2] = table_v[2, pl.ds(row0 + g * _L, _L)]
    lax.fori_loop(0, _L // _RPI, row_body, g)
    return carry

  state_q16 = [None, None, None]
  state_q = [None] * _RPI
  lax.fori_loop(0, _ROWS_PER_W // _L, group_body, 0)

  pltpu.sync_copy(out_v, out_hbm.at[pl.ds(wid * _ROWS_PER_W, _ROWS_PER_W)])


def kernel(features, points):
  del features
  pts_t = jnp.transpose(points, (0, 2, 1))
  mesh = plsc.VectorSubcoreMesh(core_axis_name="c", subcore_axis_name="s")
  kfn = pl.kernel(
      _knn_kernel,
      out_type=jax.ShapeDtypeStruct((_B * _N, _K), jnp.int32),
      mesh=mesh,
      scratch_types=[
          pltpu.VMEM((3, _N), jnp.float32),
          pltpu.VMEM((_ROWS_PER_W, _K), jnp.int32),
      ],
      compiler_params=pltpu.CompilerParams(needs_layout_passes=False),
  )
  topk = kfn(pts_t).reshape(_B, _N, _K)
  batch_idx = jnp.broadcast_to(
      jnp.arange(_B, dtype=jnp.int32).reshape(_B, 1, 1, 1), (_B, _N, _K, 1))
  return jnp.concatenate([batch_idx, topk[..., None]], axis=3)
